# baseline (speedup 1.0000x reference)
import jax
import jax.numpy as jnp
from jax.experimental import pallas as pl
from jax.experimental.pallas import tpu as pltpu


def _body(x_ref, out_ref):
    i = pl.program_id(0)

    @pl.when(i == 0)
    def _():
        out_ref[0, 0] = jnp.float32(0.0)

    out_ref[0, 0] += jnp.sum(x_ref[...])


def kernel(scores, labels, k):
    b, n = scores.shape
    flat_rows = (b * n) // 128
    xf = scores.reshape(flat_rows, 128)
    g = 125
    blk = flat_rows // g
    out = pl.pallas_call(
        _body,
        grid=(g,),
        in_specs=[pl.BlockSpec((blk, 128), lambda i: (i, 0))],
        out_specs=pl.BlockSpec((1, 1), lambda i: (0, 0),
                               memory_space=pltpu.SMEM),
        out_shape=jax.ShapeDtypeStruct((1, 1), jnp.float32),
        compiler_params=pltpu.CompilerParams(
            dimension_semantics=("arbitrary",)),
    )(xf)
    return out[0, 0]


# speedup vs baseline: 4.3218x; 4.3218x over previous
import jax
import jax.numpy as jnp
from jax import lax
from jax.experimental import pallas as pl
from jax.experimental.pallas import tpu as pltpu

_EPS = 1e-10


def _pick_chunk(n):
    for d in range(min(n, 1024), 7, -1):
        if n % d == 0 and d % 8 == 0:
            return d
    return n


def _make_body(n, b, cb, sub):
    g = n // cb
    num_sub = cb // sub
    neg_inf = float('-inf')

    def body(kmin_ref, lab_ref, x_ref, hbm_ref, out_ref,
             m1_ref, m2_ref, m3_ref, sl_ref, buf_ref, sem):
        j = pl.program_id(0)
        lab = lab_ref[...]
        rows = jax.lax.broadcasted_iota(jnp.int32, (sub, 1), 0)

        @pl.when(j == 0)
        def _():
            m1_ref[...] = jnp.full((sub, b), neg_inf)
            m2_ref[...] = jnp.full((sub, b), neg_inf)
            m3_ref[...] = jnp.full((sub, b), neg_inf)
            sl_ref[...] = jnp.full((1, b), neg_inf)

        m1 = m1_ref[...]
        m2 = m2_ref[...]
        m3 = m3_ref[...]
        sl = sl_ref[...]
        base = j * cb
        for jj in range(num_sub):
            v = x_ref[jj * sub:(jj + 1) * sub, :]
            rid = rows + (base + jj * sub)
            hit = rid == lab
            sl = jnp.maximum(sl, jnp.max(jnp.where(hit, v, neg_inf),
                                         axis=0, keepdims=True))
            t1 = jnp.maximum(m1, v)
            u1 = jnp.minimum(m1, v)
            t2 = jnp.maximum(m2, u1)
            u2 = jnp.minimum(m2, u1)
            m3 = jnp.maximum(m3, u2)
            m1, m2 = t1, t2
        m1_ref[...] = m1
        m2_ref[...] = m2
        m3_ref[...] = m3
        sl_ref[...] = sl

        @pl.when(j == g - 1)
        def _():
            z = jnp.concatenate([m1, m2, m3], axis=0)

            s1 = jnp.max(z, axis=0, keepdims=True)
            eq1 = z == s1
            cnt1 = jnp.sum(eq1.astype(jnp.int32), axis=0, keepdims=True)
            v2 = jnp.max(jnp.where(eq1, neg_inf, z), axis=0, keepdims=True)
            cnt2 = jnp.sum((z == v2).astype(jnp.int32), axis=0, keepdims=True)
            v3 = jnp.max(jnp.where(z >= v2, neg_inf, z), axis=0, keepdims=True)
            s2 = jnp.where(cnt1 >= 2, s1, v2)
            s3 = jnp.where(cnt1 >= 3, s1,
                           jnp.where(cnt1 + cnt2 >= 3, v2, v3))

            zgt = jnp.sum((z > sl).astype(jnp.int32), axis=0, keepdims=True)
            zeq = jnp.sum((z == sl).astype(jnp.int32), axis=0, keepdims=True)
            need = (zgt <= 2) & ((zeq >= 2) | (zgt + zeq >= 3))
            anyneed = jnp.any(need)

            def rescan(c, carry):
                gt, tie = carry
                cp = pltpu.make_async_copy(
                    hbm_ref.at[pl.ds(c * cb, cb)], buf_ref, sem)
                cp.start()
                cp.wait()
                cbase = c * cb
                for jj in range(num_sub):
                    v = buf_ref[jj * sub:(jj + 1) * sub, :]
                    rid = rows + (cbase + jj * sub)
                    gt = gt + jnp.sum((v > sl).astype(jnp.int32),
                                      axis=0, keepdims=True)
                    tie = tie + jnp.sum(((v == sl) & (rid < lab))
                                        .astype(jnp.int32),
                                        axis=0, keepdims=True)
                return gt, tie

            zero = jnp.zeros((1, b), jnp.int32)
            gt_x, tie_x = lax.fori_loop(
                0, jnp.where(anyneed, g, 0), rescan, (zero, zero))
            rank = jnp.where(anyneed, gt_x + tie_x, zgt)

            c1 = jnp.exp(s1)
            c2 = c1 + jnp.exp(s2)
            c3 = c2 + jnp.exp(s3)
            csel = jnp.where(rank == 0, c1, jnp.where(rank == 1, c2, c3))
            logd = jnp.log(csel + jnp.float32(1e-10))
            kmin = jnp.minimum(kmin_ref[0, 0], 3)
            contrib = jnp.where(rank < kmin, logd - sl, 0.0)
            out_ref[0, 0] = jnp.sum(contrib)

    return body, g


def kernel(scores, labels, k):
    b, n = scores.shape
    xt = scores.T
    labels2 = labels.astype(jnp.int32).reshape(1, b)
    kmin = jnp.asarray(k, jnp.int32).reshape(1, 1)

    cb = _pick_chunk(n)
    sub = 8 if (cb % 8 == 0 and cb > 8) else cb
    body, g = _make_body(n, b, cb, sub)

    loss_sum = pl.pallas_call(
        body,
        grid=(g,),
        in_specs=[
            pl.BlockSpec((1, 1), lambda i: (0, 0), memory_space=pltpu.SMEM),
            pl.BlockSpec((1, b), lambda i: (0, 0)),
            pl.BlockSpec((cb, b), lambda i: (i, 0)),
            pl.BlockSpec(memory_space=pl.ANY),
        ],
        out_specs=pl.BlockSpec((1, 1), lambda i: (0, 0),
                               memory_space=pltpu.SMEM),
        out_shape=jax.ShapeDtypeStruct((1, 1), jnp.float32),
        scratch_shapes=[
            pltpu.VMEM((sub, b), jnp.float32),
            pltpu.VMEM((sub, b), jnp.float32),
            pltpu.VMEM((sub, b), jnp.float32),
            pltpu.VMEM((1, b), jnp.float32),
            pltpu.VMEM((cb, b), jnp.float32),
            pltpu.SemaphoreType.DMA,
        ],
        compiler_params=pltpu.CompilerParams(
            dimension_semantics=("arbitrary",)),
    )(kmin, labels2, xt, xt)

    return loss_sum[0, 0] / jnp.float32(b)


# speedup vs baseline: 4.3909x; 1.0160x over previous
import jax
import jax.numpy as jnp
from jax import lax
from jax.experimental import pallas as pl
from jax.experimental.pallas import tpu as pltpu

_EPS = 1e-10


def _pick_chunk(n):
    for d in range(min(n, 1024), 7, -1):
        if n % d == 0 and d % 8 == 0:
            return d
    return n


def _make_body(n, b, cb, sub, rcb):
    g = n // cb
    rg = n // rcb
    num_sub = cb // sub
    rnum_sub = rcb // sub
    neg_inf = float('-inf')

    def body(kmin_ref, lab_ref, x_ref, hbm_ref, out_ref,
             m1_ref, m2_ref, m3_ref, sl_ref, buf_ref, sem):
        j = pl.program_id(0)
        lab = lab_ref[...]
        rows = jax.lax.broadcasted_iota(jnp.int32, (sub, 1), 0)

        @pl.when(j == 0)
        def _():
            m1_ref[...] = jnp.full((sub, b), neg_inf)
            m2_ref[...] = jnp.full((sub, b), neg_inf)
            m3_ref[...] = jnp.full((sub, b), neg_inf)
            sl_ref[...] = jnp.full((1, b), neg_inf)

        m1 = m1_ref[...]
        m2 = m2_ref[...]
        m3 = m3_ref[...]
        sl = sl_ref[...]
        base = j * cb
        for jj in range(num_sub):
            v = x_ref[jj * sub:(jj + 1) * sub, :]
            rid = rows + (base + jj * sub)
            hit = rid == lab
            sl = jnp.maximum(sl, jnp.max(jnp.where(hit, v, neg_inf),
                                         axis=0, keepdims=True))
            t1 = jnp.maximum(m1, v)
            u1 = jnp.minimum(m1, v)
            t2 = jnp.maximum(m2, u1)
            u2 = jnp.minimum(m2, u1)
            m3 = jnp.maximum(m3, u2)
            m1, m2 = t1, t2
        m1_ref[...] = m1
        m2_ref[...] = m2
        m3_ref[...] = m3
        sl_ref[...] = sl

        @pl.when(j == g - 1)
        def _():
            z = jnp.concatenate([m1, m2, m3], axis=0)

            s1 = jnp.max(z, axis=0, keepdims=True)
            eq1 = z == s1
            cnt1 = jnp.sum(eq1.astype(jnp.int32), axis=0, keepdims=True)
            v2 = jnp.max(jnp.where(eq1, neg_inf, z), axis=0, keepdims=True)
            cnt2 = jnp.sum((z == v2).astype(jnp.int32), axis=0, keepdims=True)
            v3 = jnp.max(jnp.where(z >= v2, neg_inf, z), axis=0, keepdims=True)
            s2 = jnp.where(cnt1 >= 2, s1, v2)
            s3 = jnp.where(cnt1 >= 3, s1,
                           jnp.where(cnt1 + cnt2 >= 3, v2, v3))

            zgt = jnp.sum((z > sl).astype(jnp.int32), axis=0, keepdims=True)
            zeq = jnp.sum((z == sl).astype(jnp.int32), axis=0, keepdims=True)
            need = (zgt <= 2) & ((zeq >= 2) | (zgt + zeq >= 3))
            anyneed = jnp.any(need)

            def rescan(c, carry):
                gt, tie = carry
                cp = pltpu.make_async_copy(
                    hbm_ref.at[pl.ds(c * rcb, rcb)], buf_ref, sem)
                cp.start()
                cp.wait()
                cbase = c * rcb
                for jj in range(rnum_sub):
                    v = buf_ref[jj * sub:(jj + 1) * sub, :]
                    rid = rows + (cbase + jj * sub)
                    gt = gt + jnp.sum((v > sl).astype(jnp.int32),
                                      axis=0, keepdims=True)
                    tie = tie + jnp.sum(((v == sl) & (rid < lab))
                                        .astype(jnp.int32),
                                        axis=0, keepdims=True)
                return gt, tie

            zero = jnp.zeros((1, b), jnp.int32)
            gt_x, tie_x = lax.fori_loop(
                0, jnp.where(anyneed, rg, 0), rescan, (zero, zero))
            rank = jnp.where(anyneed, gt_x + tie_x, zgt)

            c1 = jnp.exp(s1)
            c2 = c1 + jnp.exp(s2)
            c3 = c2 + jnp.exp(s3)
            csel = jnp.where(rank == 0, c1, jnp.where(rank == 1, c2, c3))
            logd = jnp.log(csel + jnp.float32(1e-10))
            kmin = jnp.minimum(kmin_ref[0, 0], 3)
            contrib = jnp.where(rank < kmin, logd - sl, 0.0)
            out_ref[0, 0] = jnp.sum(contrib)

    return body, g


def kernel(scores, labels, k):
    b, n = scores.shape
    xt = scores.T
    labels2 = labels.astype(jnp.int32).reshape(1, b)
    kmin = jnp.asarray(k, jnp.int32).reshape(1, 1)

    cb = _pick_chunk(n)
    if n % (2 * cb) == 0 and 2 * cb * b * 4 <= 2**23:
        rcb, cb = cb, 2 * cb
    else:
        rcb = cb
    sub = 8 if (cb % 8 == 0 and cb > 8) else cb
    body, g = _make_body(n, b, cb, sub, rcb)

    loss_sum = pl.pallas_call(
        body,
        grid=(g,),
        in_specs=[
            pl.BlockSpec((1, 1), lambda i: (0, 0), memory_space=pltpu.SMEM),
            pl.BlockSpec((1, b), lambda i: (0, 0)),
            pl.BlockSpec((cb, b), lambda i: (i, 0)),
            pl.BlockSpec(memory_space=pl.ANY),
        ],
        out_specs=pl.BlockSpec((1, 1), lambda i: (0, 0),
                               memory_space=pltpu.SMEM),
        out_shape=jax.ShapeDtypeStruct((1, 1), jnp.float32),
        scratch_shapes=[
            pltpu.VMEM((sub, b), jnp.float32),
            pltpu.VMEM((sub, b), jnp.float32),
            pltpu.VMEM((sub, b), jnp.float32),
            pltpu.VMEM((1, b), jnp.float32),
            pltpu.VMEM((rcb, b), jnp.float32),
            pltpu.SemaphoreType.DMA,
        ],
        compiler_params=pltpu.CompilerParams(
            dimension_semantics=("arbitrary",)),
    )(kmin, labels2, xt, xt)

    return loss_sum[0, 0] / jnp.float32(b)


# speedup vs baseline: 6.2239x; 1.4175x over previous
import jax
import jax.numpy as jnp
from jax import lax
from jax.experimental import pallas as pl
from jax.experimental.pallas import tpu as pltpu

_EPS = 1e-10


def _pick_chunk(n):
    for d in range(min(n, 1024), 7, -1):
        if n % d == 0 and d % 8 == 0:
            return d
    return n


def _make_body(n, b, cb, sub, rcb):
    g = n // cb
    rg = n // rcb
    num_sub = cb // sub
    rnum_sub = rcb // sub
    neg_inf = float('-inf')

    def body(kmin_ref, lab_ref, x_ref, hbm_ref, out_ref,
             m1_ref, m2_ref, m3_ref, sl_ref, buf_ref, sem):
        j = pl.program_id(0)
        lab = lab_ref[...]
        rows = jax.lax.broadcasted_iota(jnp.int32, (sub, 1), 0)

        @pl.when(j == 0)
        def _():
            m1_ref[...] = jnp.full((sub, b), neg_inf)
            m2_ref[...] = jnp.full((sub, b), neg_inf)
            m3_ref[...] = jnp.full((sub, b), neg_inf)
            sl_ref[...] = jnp.full((sub, b), neg_inf)

        m1 = m1_ref[...]
        m2 = m2_ref[...]
        m3 = m3_ref[...]
        slacc = sl_ref[...]
        base = j * cb
        for jj in range(num_sub):
            v = x_ref[jj * sub:(jj + 1) * sub, :]
            rid = rows + (base + jj * sub)
            hit = rid == lab
            slacc = jnp.where(hit, v, slacc)
            t1 = jnp.maximum(m1, v)
            u1 = jnp.minimum(m1, v)
            t2 = jnp.maximum(m2, u1)
            u2 = jnp.minimum(m2, u1)
            m3 = jnp.maximum(m3, u2)
            m1, m2 = t1, t2
        m1_ref[...] = m1
        m2_ref[...] = m2
        m3_ref[...] = m3
        sl_ref[...] = slacc

        @pl.when(j == g - 1)
        def _():
            sl = jnp.max(slacc, axis=0, keepdims=True)
            z = jnp.concatenate([m1, m2, m3], axis=0)

            s1 = jnp.max(z, axis=0, keepdims=True)
            eq1 = z == s1
            cnt1 = jnp.sum(eq1.astype(jnp.int32), axis=0, keepdims=True)
            v2 = jnp.max(jnp.where(eq1, neg_inf, z), axis=0, keepdims=True)
            cnt2 = jnp.sum((z == v2).astype(jnp.int32), axis=0, keepdims=True)
            v3 = jnp.max(jnp.where(z >= v2, neg_inf, z), axis=0, keepdims=True)
            s2 = jnp.where(cnt1 >= 2, s1, v2)
            s3 = jnp.where(cnt1 >= 3, s1,
                           jnp.where(cnt1 + cnt2 >= 3, v2, v3))

            zgt = jnp.sum((z > sl).astype(jnp.int32), axis=0, keepdims=True)
            zeq = jnp.sum((z == sl).astype(jnp.int32), axis=0, keepdims=True)
            need = (zgt <= 2) & ((zeq >= 2) | (zgt + zeq >= 3))
            anyneed = jnp.any(need)

            def rescan(c, carry):
                gt, tie = carry
                cp = pltpu.make_async_copy(
                    hbm_ref.at[pl.ds(c * rcb, rcb)], buf_ref, sem)
                cp.start()
                cp.wait()
                cbase = c * rcb
                for jj in range(rnum_sub):
                    v = buf_ref[jj * sub:(jj + 1) * sub, :]
                    rid = rows + (cbase + jj * sub)
                    gt = gt + jnp.sum((v > sl).astype(jnp.int32),
                                      axis=0, keepdims=True)
                    tie = tie + jnp.sum(((v == sl) & (rid < lab))
                                        .astype(jnp.int32),
                                        axis=0, keepdims=True)
                return gt, tie

            zero = jnp.zeros((1, b), jnp.int32)
            gt_x, tie_x = lax.fori_loop(
                0, jnp.where(anyneed, rg, 0), rescan, (zero, zero))
            rank = jnp.where(anyneed, gt_x + tie_x, zgt)

            c1 = jnp.exp(s1)
            c2 = c1 + jnp.exp(s2)
            c3 = c2 + jnp.exp(s3)
            csel = jnp.where(rank == 0, c1, jnp.where(rank == 1, c2, c3))
            logd = jnp.log(csel + jnp.float32(1e-10))
            kmin = jnp.minimum(kmin_ref[0, 0], 3)
            contrib = jnp.where(rank < kmin, logd - sl, 0.0)
            out_ref[0, 0] = jnp.sum(contrib)

    return body, g


def kernel(scores, labels, k):
    b, n = scores.shape
    xt = scores.T
    labels2 = labels.astype(jnp.int32).reshape(1, b)
    kmin = jnp.asarray(k, jnp.int32).reshape(1, 1)

    cb = _pick_chunk(n)
    if n % (2 * cb) == 0 and 2 * cb * b * 4 <= 2**23:
        rcb, cb = cb, 2 * cb
    else:
        rcb = cb
    sub = 8 if (cb % 8 == 0 and cb > 8) else cb
    body, g = _make_body(n, b, cb, sub, rcb)

    loss_sum = pl.pallas_call(
        body,
        grid=(g,),
        in_specs=[
            pl.BlockSpec((1, 1), lambda i: (0, 0), memory_space=pltpu.SMEM),
            pl.BlockSpec((1, b), lambda i: (0, 0)),
            pl.BlockSpec((cb, b), lambda i: (i, 0)),
            pl.BlockSpec(memory_space=pl.ANY),
        ],
        out_specs=pl.BlockSpec((1, 1), lambda i: (0, 0),
                               memory_space=pltpu.SMEM),
        out_shape=jax.ShapeDtypeStruct((1, 1), jnp.float32),
        scratch_shapes=[
            pltpu.VMEM((sub, b), jnp.float32),
            pltpu.VMEM((sub, b), jnp.float32),
            pltpu.VMEM((sub, b), jnp.float32),
            pltpu.VMEM((sub, b), jnp.float32),
            pltpu.VMEM((rcb, b), jnp.float32),
            pltpu.SemaphoreType.DMA,
        ],
        compiler_params=pltpu.CompilerParams(
            dimension_semantics=("arbitrary",)),
    )(kmin, labels2, xt, xt)

    return loss_sum[0, 0] / jnp.float32(b)
